# baseline (speedup 1.0000x reference)
import jax
import jax.numpy as jnp
from jax.experimental import pallas as pl
from jax.experimental.pallas import tpu as pltpu

B, H, W = 4, 384, 384
CIN, CF, NCLS = 3, 96, 10
MAXLEN = 512
NTOK = 512
ROWS = 32
P = ROWS * W
NT = H // ROWS


def _main_body(img_ref, gts_ref, seg_ref, w1_ref, w2_ref,
               cnn_ref, segout_ref, tok_ref, lbl_ref,
               acc_ref):
    b = pl.program_id(0)
    t = pl.program_id(1)

    @pl.when(t == 0)
    def _init():
        acc_ref[...] = jnp.zeros_like(acc_ref)

    img_r = img_ref[0].reshape(CIN, P)
    feats_t = jax.nn.relu(
        jax.lax.dot_general(w1_ref[...], img_r,
                            (((0,), (0,)), ((), ())),
                            preferred_element_type=jnp.float32))
    cnn = jax.lax.dot_general(w2_ref[...], feats_t,
                              (((0,), (0,)), ((), ())),
                              preferred_element_type=jnp.float32)
    cnn_ref[0] = cnn.reshape(NCLS, ROWS, W)

    seg = seg_ref[0]
    segout_ref[0] = seg + (b * NTOK + 1)

    seg_row = seg.reshape(1, P).astype(jnp.int16)
    iota_s = jax.lax.broadcasted_iota(jnp.int32, (NTOK, 1), 0).astype(jnp.int16)
    onehot = (iota_s == seg_row).astype(jnp.bfloat16)

    gts_row = gts_ref[0].reshape(1, P).astype(jnp.int16)
    iota_c = jax.lax.broadcasted_iota(jnp.int32, (NCLS, 1), 0).astype(jnp.int16)
    cls_oh = (iota_c == gts_row).astype(jnp.bfloat16)

    feats_b = feats_t.astype(jnp.bfloat16)
    rhs = jnp.concatenate([feats_b, cls_oh], axis=0)
    acc_ref[...] += jax.lax.dot_general(
        onehot, rhs, (((1,), (1,)), ((), ())),
        preferred_element_type=jnp.float32)

    @pl.when(t == NT - 1)
    def _finalize():
        cls = acc_ref[:, CF:CF + NCLS]
        counts = jnp.sum(cls, axis=1, keepdims=True)
        tok_ref[0] = acc_ref[:, :CF] / jnp.maximum(counts, 1.0)
        mx = jnp.max(cls, axis=1, keepdims=True)
        idx = jax.lax.broadcasted_iota(jnp.int32, (NTOK, NCLS), 1)
        first = jnp.min(jnp.where(cls == mx, idx, NCLS), axis=1)
        lbl_ref[0] = first.astype(jnp.float32).reshape(1, NTOK)


def _attn_body(tok_ref, valid_ref, wq_ref, wk_ref, wv_ref, wo_ref, out_ref):
    tok = tok_ref[0]
    q = jnp.dot(tok, wq_ref[...], preferred_element_type=jnp.float32)
    k = jnp.dot(tok, wk_ref[...], preferred_element_type=jnp.float32)
    v = jnp.dot(tok, wv_ref[...], preferred_element_type=jnp.float32)
    s = jax.lax.dot_general(q, k, (((1,), (1,)), ((), ())),
                            preferred_element_type=jnp.float32)
    s = s * (1.0 / jnp.sqrt(jnp.float32(CF)))
    m = jnp.max(s, axis=1, keepdims=True)
    e = jnp.exp(s - m)
    p = e / jnp.sum(e, axis=1, keepdims=True)
    valid = valid_ref[0, 0]
    p = p * valid.reshape(MAXLEN, 1) * valid.reshape(1, MAXLEN)
    ctx = jnp.dot(p, v, preferred_element_type=jnp.float32)
    out_ref[0] = jnp.dot(ctx, wo_ref[...],
                         preferred_element_type=jnp.float32)


def kernel(img, gts, segments, n_tokens, W1, W2, Wq, Wk, Wv, Wo):
    cnn_logits, seg_global, tokens, super_labels = pl.pallas_call(
        _main_body,
        grid=(B, NT),
        in_specs=[
            pl.BlockSpec((1, CIN, ROWS, W), lambda b, t: (b, 0, t, 0)),
            pl.BlockSpec((1, ROWS, W), lambda b, t: (b, t, 0)),
            pl.BlockSpec((1, ROWS, W), lambda b, t: (b, t, 0)),
            pl.BlockSpec((CIN, CF), lambda b, t: (0, 0)),
            pl.BlockSpec((CF, NCLS), lambda b, t: (0, 0)),
        ],
        out_specs=[
            pl.BlockSpec((1, NCLS, ROWS, W), lambda b, t: (b, 0, t, 0)),
            pl.BlockSpec((1, ROWS, W), lambda b, t: (b, t, 0)),
            pl.BlockSpec((1, NTOK, CF), lambda b, t: (b, 0, 0)),
            pl.BlockSpec((1, 1, NTOK), lambda b, t: (b, 0, 0)),
        ],
        out_shape=[
            jax.ShapeDtypeStruct((B, NCLS, H, W), jnp.float32),
            jax.ShapeDtypeStruct((B, H, W), jnp.int32),
            jax.ShapeDtypeStruct((B, NTOK, CF), jnp.float32),
            jax.ShapeDtypeStruct((B, 1, NTOK), jnp.float32),
        ],
        scratch_shapes=[
            pltpu.VMEM((NTOK, CF + NCLS), jnp.float32),
        ],
    )(img, gts, segments, W1, W2)

    super_labels = super_labels.reshape(B, NTOK)
    valid = (jnp.arange(MAXLEN)[None, :] < n_tokens[:, None]).astype(jnp.float32)

    trans_logits = pl.pallas_call(
        _attn_body,
        grid=(B,),
        in_specs=[
            pl.BlockSpec((1, MAXLEN, CF), lambda b: (b, 0, 0)),
            pl.BlockSpec((1, 1, MAXLEN), lambda b: (b, 0, 0)),
            pl.BlockSpec((CF, CF), lambda b: (0, 0)),
            pl.BlockSpec((CF, CF), lambda b: (0, 0)),
            pl.BlockSpec((CF, CF), lambda b: (0, 0)),
            pl.BlockSpec((CF, NCLS), lambda b: (0, 0)),
        ],
        out_specs=pl.BlockSpec((1, MAXLEN, NCLS), lambda b: (b, 0, 0)),
        out_shape=jax.ShapeDtypeStruct((B, MAXLEN, NCLS), jnp.float32),
    )(tokens, valid.reshape(B, 1, MAXLEN), Wq, Wk, Wv, Wo)

    tokens_ids = jnp.arange(1, B * NTOK + 1)
    return (cnn_logits, trans_logits, super_labels, valid, tokens_ids,
            seg_global)


# speedup vs baseline: 1.2265x; 1.2265x over previous
import jax
import jax.numpy as jnp
from jax.experimental import pallas as pl
from jax.experimental.pallas import tpu as pltpu

B, H, W = 4, 384, 384
CIN, CF, NCLS = 3, 96, 10
MAXLEN = 512
NTOK = 512
ROWS = 32
P = ROWS * W
NT = H // ROWS


def _main_body(img_ref, gts_ref, seg_ref, w1_ref, w2_ref,
               cnn_ref, segout_ref, tok_ref, lbl_ref,
               acc_ref):
    b = pl.program_id(0)
    t = pl.program_id(1)

    @pl.when(t == 0)
    def _init():
        acc_ref[...] = jnp.zeros_like(acc_ref)

    img_r = img_ref[0].reshape(CIN, P).astype(jnp.bfloat16)
    feats_t = jax.nn.relu(
        jax.lax.dot_general(w1_ref[...].astype(jnp.bfloat16), img_r,
                            (((0,), (0,)), ((), ())),
                            preferred_element_type=jnp.float32))
    cnn = jax.lax.dot_general(w2_ref[...], feats_t,
                              (((0,), (0,)), ((), ())),
                              preferred_element_type=jnp.float32)
    cnn_ref[0] = cnn.reshape(NCLS, ROWS, W)

    seg = seg_ref[0]
    segout_ref[0] = seg + (b * NTOK + 1)

    seg_row = seg.reshape(1, P)
    iota_s = jax.lax.broadcasted_iota(jnp.int32, (NTOK, 1), 0)
    onehot = (iota_s == seg_row).astype(jnp.bfloat16)

    gts_row = gts_ref[0].reshape(1, P)
    iota_c = jax.lax.broadcasted_iota(jnp.int32, (NCLS, 1), 0)
    cls_oh = (iota_c == gts_row).astype(jnp.bfloat16)

    feats_b = feats_t.astype(jnp.bfloat16)
    rhs = jnp.concatenate([feats_b, cls_oh], axis=0)
    acc_ref[...] += jax.lax.dot_general(
        onehot, rhs, (((1,), (1,)), ((), ())),
        preferred_element_type=jnp.float32)

    @pl.when(t == NT - 1)
    def _finalize():
        cls = acc_ref[:, CF:CF + NCLS]
        counts = jnp.sum(cls, axis=1, keepdims=True)
        tok_ref[0] = acc_ref[:, :CF] / jnp.maximum(counts, 1.0)
        mx = jnp.max(cls, axis=1, keepdims=True)
        idx = jax.lax.broadcasted_iota(jnp.int32, (NTOK, NCLS), 1)
        first = jnp.min(jnp.where(cls == mx, idx, NCLS), axis=1)
        lbl_ref[0] = first.astype(jnp.float32).reshape(1, NTOK)


def _attn_body(tok_ref, valid_ref, wq_ref, wk_ref, wv_ref, wo_ref, out_ref):
    tok = tok_ref[0]
    q = jnp.dot(tok, wq_ref[...], preferred_element_type=jnp.float32)
    k = jnp.dot(tok, wk_ref[...], preferred_element_type=jnp.float32)
    v = jnp.dot(tok, wv_ref[...], preferred_element_type=jnp.float32)
    s = jax.lax.dot_general(q, k, (((1,), (1,)), ((), ())),
                            preferred_element_type=jnp.float32)
    s = s * (1.0 / jnp.sqrt(jnp.float32(CF)))
    m = jnp.max(s, axis=1, keepdims=True)
    e = jnp.exp(s - m)
    p = e / jnp.sum(e, axis=1, keepdims=True)
    valid = valid_ref[0, 0]
    p = p * valid.reshape(MAXLEN, 1) * valid.reshape(1, MAXLEN)
    ctx = jnp.dot(p, v, preferred_element_type=jnp.float32)
    out_ref[0] = jnp.dot(ctx, wo_ref[...],
                         preferred_element_type=jnp.float32)


def kernel(img, gts, segments, n_tokens, W1, W2, Wq, Wk, Wv, Wo):
    cnn_logits, seg_global, tokens, super_labels = pl.pallas_call(
        _main_body,
        grid=(B, NT),
        in_specs=[
            pl.BlockSpec((1, CIN, ROWS, W), lambda b, t: (b, 0, t, 0)),
            pl.BlockSpec((1, ROWS, W), lambda b, t: (b, t, 0)),
            pl.BlockSpec((1, ROWS, W), lambda b, t: (b, t, 0)),
            pl.BlockSpec((CIN, CF), lambda b, t: (0, 0)),
            pl.BlockSpec((CF, NCLS), lambda b, t: (0, 0)),
        ],
        out_specs=[
            pl.BlockSpec((1, NCLS, ROWS, W), lambda b, t: (b, 0, t, 0)),
            pl.BlockSpec((1, ROWS, W), lambda b, t: (b, t, 0)),
            pl.BlockSpec((1, NTOK, CF), lambda b, t: (b, 0, 0)),
            pl.BlockSpec((1, 1, NTOK), lambda b, t: (b, 0, 0)),
        ],
        out_shape=[
            jax.ShapeDtypeStruct((B, NCLS, H, W), jnp.float32),
            jax.ShapeDtypeStruct((B, H, W), jnp.int32),
            jax.ShapeDtypeStruct((B, NTOK, CF), jnp.float32),
            jax.ShapeDtypeStruct((B, 1, NTOK), jnp.float32),
        ],
        scratch_shapes=[
            pltpu.VMEM((NTOK, CF + NCLS), jnp.float32),
        ],
    )(img, gts, segments, W1, W2)

    super_labels = super_labels.reshape(B, NTOK)
    valid = (jnp.arange(MAXLEN)[None, :] < n_tokens[:, None]).astype(jnp.float32)

    trans_logits = pl.pallas_call(
        _attn_body,
        grid=(B,),
        in_specs=[
            pl.BlockSpec((1, MAXLEN, CF), lambda b: (b, 0, 0)),
            pl.BlockSpec((1, 1, MAXLEN), lambda b: (b, 0, 0)),
            pl.BlockSpec((CF, CF), lambda b: (0, 0)),
            pl.BlockSpec((CF, CF), lambda b: (0, 0)),
            pl.BlockSpec((CF, CF), lambda b: (0, 0)),
            pl.BlockSpec((CF, NCLS), lambda b: (0, 0)),
        ],
        out_specs=pl.BlockSpec((1, MAXLEN, NCLS), lambda b: (b, 0, 0)),
        out_shape=jax.ShapeDtypeStruct((B, MAXLEN, NCLS), jnp.float32),
    )(tokens, valid.reshape(B, 1, MAXLEN), Wq, Wk, Wv, Wo)

    tokens_ids = jnp.arange(1, B * NTOK + 1)
    return (cnn_logits, trans_logits, super_labels, valid, tokens_ids,
            seg_global)


# speedup vs baseline: 1.3648x; 1.1128x over previous
import jax
import jax.numpy as jnp
from jax.experimental import pallas as pl
from jax.experimental.pallas import tpu as pltpu

B, H, W = 4, 384, 384
CIN, CF, NCLS = 3, 96, 10
MAXLEN = 512
NTOK = 512
ROWS = 32
P = ROWS * W
NT = H // ROWS


def _main_body(img_ref, gts_ref, seg_ref, w1_ref, w2_ref,
               cnn_ref, segout_ref, tok_ref, lbl_ref,
               acc_ref):
    b = pl.program_id(0)
    t = pl.program_id(1)

    @pl.when(t == 0)
    def _init():
        acc_ref[...] = jnp.zeros_like(acc_ref)

    img_r = img_ref[0].reshape(CIN, P).astype(jnp.bfloat16)
    feats_t = jax.nn.relu(
        jax.lax.dot_general(w1_ref[...].astype(jnp.bfloat16), img_r,
                            (((0,), (0,)), ((), ())),
                            preferred_element_type=jnp.float32))
    cnn = jax.lax.dot_general(w2_ref[...], feats_t,
                              (((0,), (0,)), ((), ())),
                              preferred_element_type=jnp.float32)
    cnn_ref[0] = cnn.reshape(NCLS, ROWS, W)

    seg = seg_ref[0]
    segout_ref[0] = seg + (b * NTOK + 1)

    seg_row = seg.reshape(1, P)
    iota_s = jax.lax.broadcasted_iota(jnp.int32, (NTOK, 1), 0)
    onehot = (iota_s == seg_row).astype(jnp.float8_e4m3fn)

    gts_row = gts_ref[0].reshape(1, P)
    iota_c = jax.lax.broadcasted_iota(jnp.int32, (NCLS, 1), 0)
    cls_oh = (iota_c == gts_row).astype(jnp.float8_e4m3fn)

    feats_b = feats_t.astype(jnp.float8_e4m3fn)
    rhs = jnp.concatenate([feats_b, cls_oh], axis=0)
    acc_ref[...] += jax.lax.dot_general(
        onehot, rhs, (((1,), (1,)), ((), ())),
        preferred_element_type=jnp.float32)

    @pl.when(t == NT - 1)
    def _finalize():
        cls = acc_ref[:, CF:CF + NCLS]
        counts = jnp.sum(cls, axis=1, keepdims=True)
        tok_ref[0] = acc_ref[:, :CF] / jnp.maximum(counts, 1.0)
        mx = jnp.max(cls, axis=1, keepdims=True)
        idx = jax.lax.broadcasted_iota(jnp.int32, (NTOK, NCLS), 1)
        first = jnp.min(jnp.where(cls == mx, idx, NCLS), axis=1)
        lbl_ref[0] = first.astype(jnp.float32).reshape(1, NTOK)


def _attn_body(tok_ref, valid_ref, wq_ref, wk_ref, wv_ref, wo_ref, out_ref):
    tok = tok_ref[0]
    q = jnp.dot(tok, wq_ref[...], preferred_element_type=jnp.float32)
    k = jnp.dot(tok, wk_ref[...], preferred_element_type=jnp.float32)
    v = jnp.dot(tok, wv_ref[...], preferred_element_type=jnp.float32)
    s = jax.lax.dot_general(q, k, (((1,), (1,)), ((), ())),
                            preferred_element_type=jnp.float32)
    s = s * (1.0 / jnp.sqrt(jnp.float32(CF)))
    m = jnp.max(s, axis=1, keepdims=True)
    e = jnp.exp(s - m)
    p = e / jnp.sum(e, axis=1, keepdims=True)
    valid = valid_ref[0, 0]
    p = p * valid.reshape(MAXLEN, 1) * valid.reshape(1, MAXLEN)
    ctx = jnp.dot(p, v, preferred_element_type=jnp.float32)
    out_ref[0] = jnp.dot(ctx, wo_ref[...],
                         preferred_element_type=jnp.float32)


def kernel(img, gts, segments, n_tokens, W1, W2, Wq, Wk, Wv, Wo):
    cnn_logits, seg_global, tokens, super_labels = pl.pallas_call(
        _main_body,
        grid=(B, NT),
        in_specs=[
            pl.BlockSpec((1, CIN, ROWS, W), lambda b, t: (b, 0, t, 0)),
            pl.BlockSpec((1, ROWS, W), lambda b, t: (b, t, 0)),
            pl.BlockSpec((1, ROWS, W), lambda b, t: (b, t, 0)),
            pl.BlockSpec((CIN, CF), lambda b, t: (0, 0)),
            pl.BlockSpec((CF, NCLS), lambda b, t: (0, 0)),
        ],
        out_specs=[
            pl.BlockSpec((1, NCLS, ROWS, W), lambda b, t: (b, 0, t, 0)),
            pl.BlockSpec((1, ROWS, W), lambda b, t: (b, t, 0)),
            pl.BlockSpec((1, NTOK, CF), lambda b, t: (b, 0, 0)),
            pl.BlockSpec((1, 1, NTOK), lambda b, t: (b, 0, 0)),
        ],
        out_shape=[
            jax.ShapeDtypeStruct((B, NCLS, H, W), jnp.float32),
            jax.ShapeDtypeStruct((B, H, W), jnp.int32),
            jax.ShapeDtypeStruct((B, NTOK, CF), jnp.float32),
            jax.ShapeDtypeStruct((B, 1, NTOK), jnp.float32),
        ],
        scratch_shapes=[
            pltpu.VMEM((NTOK, CF + NCLS), jnp.float32),
        ],
    )(img, gts, segments, W1, W2)

    super_labels = super_labels.reshape(B, NTOK)
    valid = (jnp.arange(MAXLEN)[None, :] < n_tokens[:, None]).astype(jnp.float32)

    trans_logits = pl.pallas_call(
        _attn_body,
        grid=(B,),
        in_specs=[
            pl.BlockSpec((1, MAXLEN, CF), lambda b: (b, 0, 0)),
            pl.BlockSpec((1, 1, MAXLEN), lambda b: (b, 0, 0)),
            pl.BlockSpec((CF, CF), lambda b: (0, 0)),
            pl.BlockSpec((CF, CF), lambda b: (0, 0)),
            pl.BlockSpec((CF, CF), lambda b: (0, 0)),
            pl.BlockSpec((CF, NCLS), lambda b: (0, 0)),
        ],
        out_specs=pl.BlockSpec((1, MAXLEN, NCLS), lambda b: (b, 0, 0)),
        out_shape=jax.ShapeDtypeStruct((B, MAXLEN, NCLS), jnp.float32),
    )(tokens, valid.reshape(B, 1, MAXLEN), Wq, Wk, Wv, Wo)

    tokens_ids = jnp.arange(1, B * NTOK + 1)
    return (cnn_logits, trans_logits, super_labels, valid, tokens_ids,
            seg_global)
